# Initial kernel scaffold; baseline (speedup 1.0000x reference)
import jax
import jax.numpy as jnp
from jax.experimental import pallas as pl


def kernel(z, emb):
    raise NotImplementedError("write your pallas kernel here")


# speedup vs baseline: 1.0014x; 1.0014x over previous
import functools

import jax
import jax.numpy as jnp
from jax import lax
from jax.experimental import pallas as pl
from jax.experimental.pallas import tpu as pltpu
from jax.experimental.pallas import tpu_sc as plsc

N_EMB = 8192
EMB_DIM = 256
BETA = 10.0

_NB = 16
_BN = N_EMB // _NB

_B = 16
_HW = 1024
_NPOS = _B * _HW


def _argmin_body(zn_ref, z2_ref, emb_ref, idx_ref):
    z2_blk = z2_ref[0]
    zn = zn_ref[0]
    row = jax.lax.broadcasted_iota(jnp.int32, (_BN, _HW), 0)
    big = jnp.int32(2**30)

    def step(c, carry):
        run_val, run_idx = carry
        e_blk = emb_ref[pl.ds(c * _BN, _BN), :]
        en = jnp.sum(e_blk * e_blk, axis=1, keepdims=True)
        mm2 = jax.lax.dot_general(
            e_blk, z2_blk, (((1,), (0,)), ((), ())),
            preferred_element_type=jnp.float32)
        d = (zn + en) - mm2
        m = jnp.min(d, axis=0, keepdims=True)
        ii = jnp.min(jnp.where(d == m, row, big),
                     axis=0, keepdims=True) + c * _BN
        upd = m < run_val
        return (jnp.where(upd, m, run_val), jnp.where(upd, ii, run_idx))

    init = (jnp.full((1, _HW), jnp.inf, jnp.float32),
            jnp.zeros((1, _HW), jnp.int32))
    _, best_idx = lax.fori_loop(0, _NB, step, init)
    idx_ref[0] = best_idx


def _argmin_call(zn, z3x2, emb):
    return pl.pallas_call(
        _argmin_body,
        grid=(_B,),
        in_specs=[
            pl.BlockSpec((1, 1, _HW), lambda b: (b, 0, 0)),
            pl.BlockSpec((1, EMB_DIM, _HW), lambda b: (b, 0, 0)),
            pl.BlockSpec((N_EMB, EMB_DIM), lambda b: (0, 0)),
        ],
        out_specs=pl.BlockSpec((1, 1, _HW), lambda b: (b, 0, 0)),
        out_shape=jax.ShapeDtypeStruct((_B, 1, _HW), jnp.int32),
    )(zn, z3x2, emb)


_NC = 2
_NS = 16
_NW = _NC * _NS
_ROWS_PER_W = _NPOS // _NW
_CHUNK = 128
_NCHUNK = _ROWS_PER_W // _CHUNK


def _gather_body(emb_hbm, idx_hbm, out_hbm, idx_v, rows0, rows1, sem0, sem1):
    wid = lax.axis_index("s") * _NC + lax.axis_index("c")
    base = wid * _ROWS_PER_W
    pltpu.sync_copy(idx_hbm.at[pl.ds(wid * _NCHUNK, _NCHUNK)], idx_v)
    bufs = (rows0, rows1)
    sems = (sem0, sem1)
    cps = [pltpu.async_copy(emb_hbm.at[idx_v.at[0]], rows0, sem0)]
    for j in range(_NCHUNK):
        if j + 1 < _NCHUNK:
            cps.append(pltpu.async_copy(
                emb_hbm.at[idx_v.at[j + 1]], bufs[(j + 1) % 2], sems[(j + 1) % 2]))
        cps[j].wait()
        pltpu.sync_copy(bufs[j % 2],
                        out_hbm.at[pl.ds(base + j * _CHUNK, _CHUNK)])


def _gather_call(emb, idx):
    idx2 = idx.reshape(_NPOS // _CHUNK, _CHUNK)
    mesh = plsc.VectorSubcoreMesh(core_axis_name="c", subcore_axis_name="s")
    return pl.kernel(
        _gather_body,
        out_type=jax.ShapeDtypeStruct((_NPOS, EMB_DIM), jnp.float32),
        mesh=mesh,
        scratch_types=[
            pltpu.VMEM((_NCHUNK, _CHUNK), jnp.int32),
            pltpu.VMEM((_CHUNK, EMB_DIM), jnp.float32),
            pltpu.VMEM((_CHUNK, EMB_DIM), jnp.float32),
            pltpu.SemaphoreType.DMA,
            pltpu.SemaphoreType.DMA,
        ],
    )(emb, idx2)


def _finish_body(zq_ref, z_ref, out_ref, acc_ref):
    b = pl.program_id(0)
    zq_b = zq_ref[0]
    zq_t = zq_b.T
    z_b = z_ref[0]
    diff = zq_t - z_b
    out_ref[0] = z_b + diff
    s = jnp.sum(diff * diff)

    @pl.when(b == 0)
    def _():
        acc_ref[0, 0] = s

    @pl.when(b != 0)
    def _():
        acc_ref[0, 0] += s


def _finish_call(zq_flat, z3):
    zq4 = zq_flat.reshape(_B, _HW, EMB_DIM)
    return pl.pallas_call(
        _finish_body,
        grid=(_B,),
        in_specs=[
            pl.BlockSpec((1, _HW, EMB_DIM), lambda b: (b, 0, 0)),
            pl.BlockSpec((1, EMB_DIM, _HW), lambda b: (b, 0, 0)),
        ],
        out_specs=[
            pl.BlockSpec((1, EMB_DIM, _HW), lambda b: (b, 0, 0)),
            pl.BlockSpec(memory_space=pltpu.SMEM),
        ],
        out_shape=[
            jax.ShapeDtypeStruct((_B, EMB_DIM, _HW), jnp.float32),
            jax.ShapeDtypeStruct((1, 1), jnp.float32),
        ],
    )(zq4, z3)


def kernel(z, emb):
    B, C, H, W = z.shape
    z3 = z.reshape(B, C, H * W)

    zn = jnp.sum(jnp.transpose(z, (0, 2, 3, 1)).reshape(-1, EMB_DIM) ** 2,
                 axis=1)
    zn3 = zn.reshape(B, 1, H * W)

    idx3 = _argmin_call(zn3, z3 * 2.0, emb)
    min_encoding_indices = idx3.reshape(-1)

    zq_flat = _gather_call(emb, min_encoding_indices)

    zq_t, ssq = _finish_call(zq_flat, z3)
    m = ssq[0, 0] / jnp.float32(_NPOS * EMB_DIM)
    loss = m + jnp.float32(BETA) * m
    z_q = zq_t.reshape(B, C, H, W)
    return (z_q, loss, min_encoding_indices)
